# baseline (speedup 1.0000x reference)
import functools

import jax
import jax.numpy as jnp
from jax import lax
from jax.experimental import pallas as pl
from jax.experimental.pallas import tpu as pltpu
from jax.experimental.pallas import tpu_sc as plsc

L = 4
N = 10000
E = 320000
D = 128

NC = 2
NS = 16
NW = NC * NS

GW = 128
EBLK = 2000
NBLK = 1000
ZR = 80
NZCH = N // ZR
SPL = 4
EH = E // SPL
NCHH = EH // GW
EHBLK = EH // EBLK

@functools.cache
def _vector_mesh():
    return plsc.VectorSubcoreMesh(core_axis_name="c", subcore_axis_name="s")


def _bmm(a, w):
    return jnp.dot(a, w, preferred_element_type=jnp.float32)


def _project_body(n_ref, w_ref, out_ref):
    out_ref[0] = _bmm(n_ref[...], w_ref[0])


def _project(n, wsd):
    return pl.pallas_call(
        _project_body,
        grid=(2, N // NBLK),
        in_specs=[
            pl.BlockSpec((NBLK, D), lambda t, i: (i, 0)),
            pl.BlockSpec((1, D, D), lambda t, i: (t, 0, 0)),
        ],
        out_specs=pl.BlockSpec((1, NBLK, D), lambda t, i: (t, i, 0)),
        out_shape=jax.ShapeDtypeStruct((2, N, D), jnp.float32),
    )(n, wsd)


def _edge_body(e_ref, g1_ref, g2_ref, w1_ref, b1_ref, w2_ref, b2_ref,
               lng_ref, lnb_ref, o_ref):
    e = e_ref[...]
    pre = (_bmm(e, w1_ref[...])
           + g1_ref[...] + g2_ref[...] + b1_ref[...])
    h1 = pre * jax.nn.sigmoid(pre)
    h = _bmm(h1, w2_ref[...]) + b2_ref[...]
    m = jnp.mean(h, axis=-1, keepdims=True)
    v = jnp.mean((h - m) * (h - m), axis=-1, keepdims=True)
    hn = (h - m) * lax.rsqrt(v + 1e-5) * lng_ref[...] + lnb_ref[...]
    o_ref[...] = e + hn


def _edge_mlp(e, e_off, g, w1, b1, w2, b2, lng, lnb):
    row = lambda i: (i, 0)
    full = lambda i: (0, 0)
    return pl.pallas_call(
        _edge_body,
        grid=(EHBLK,),
        in_specs=[
            pl.BlockSpec((EBLK, D), lambda i: (e_off + i, 0)),
            pl.BlockSpec((EBLK, D), row),
            pl.BlockSpec((EBLK, D), lambda i: (EHBLK + i, 0)),
            pl.BlockSpec((D, D), full),
            pl.BlockSpec((1, D), full),
            pl.BlockSpec((D, D), full),
            pl.BlockSpec((1, D), full),
            pl.BlockSpec((1, D), full),
            pl.BlockSpec((1, D), full),
        ],
        out_specs=pl.BlockSpec((EBLK, D), row),
        out_shape=jax.ShapeDtypeStruct((EH, D), jnp.float32),
    )(e, g, g, w1, b1.reshape(1, D), w2, b2.reshape(1, D),
      lng.reshape(1, D), lnb.reshape(1, D))


def _node_body(n_ref, *rest):
    (a_refs, (w1a_ref, w1b_ref, b1_ref, w2_ref, b2_ref, lng_ref, lnb_ref,
              o_ref)) = rest[:2 * SPL], rest[2 * SPL:]
    n = n_ref[...]
    agg = a_refs[0][0]
    for r in a_refs[1:]:
        agg = agg + r[0]
    pre = (_bmm(n, w1a_ref[...]) + _bmm(agg, w1b_ref[...]) + b1_ref[...])
    h1 = pre * jax.nn.sigmoid(pre)
    h = _bmm(h1, w2_ref[...]) + b2_ref[...]
    m = jnp.mean(h, axis=-1, keepdims=True)
    v = jnp.mean((h - m) * (h - m), axis=-1, keepdims=True)
    hn = (h - m) * lax.rsqrt(v + 1e-5) * lng_ref[...] + lnb_ref[...]
    o_ref[...] = n + hn


def _node_mlp(n, aggps, w1a, w1b, b1, w2, b2, lng, lnb):
    row = lambda i: (i, 0)
    full = lambda i: (0, 0)
    p0 = lambda i: (0, i, 0)
    p1 = lambda i: (1, i, 0)
    agg_specs = []
    agg_args = []
    for a in aggps:
        agg_specs += [pl.BlockSpec((1, NBLK, D), p0),
                      pl.BlockSpec((1, NBLK, D), p1)]
        agg_args += [a, a]
    return pl.pallas_call(
        _node_body,
        grid=(N // NBLK,),
        in_specs=[pl.BlockSpec((NBLK, D), row)] + agg_specs + [
            pl.BlockSpec((D, D), full),
            pl.BlockSpec((D, D), full),
            pl.BlockSpec((1, D), full),
            pl.BlockSpec((D, D), full),
            pl.BlockSpec((1, D), full),
            pl.BlockSpec((1, D), full),
            pl.BlockSpec((1, D), full),
        ],
        out_specs=pl.BlockSpec((NBLK, D), row),
        out_shape=jax.ShapeDtypeStruct((N, D), jnp.float32),
    )(n, *agg_args, w1a, w1b, b1.reshape(1, D), w2,
      b2.reshape(1, D), lng.reshape(1, D), lnb.reshape(1, D))


@functools.cache
def _gather_kernel():
    @functools.partial(
        pl.kernel,
        out_type=jax.ShapeDtypeStruct((2 * EH, D), jnp.float32),
        mesh=_vector_mesh(),
    )
    def _gather(tab_hbm, j_hbm, g_hbm):
        def body(j_v, g_v):
            pltpu.sync_copy(tab_hbm.at[j_v.at[0]], g_v)

        pltpu.emit_pipeline(
            body,
            grid=(2 * EH // GW,),
            in_specs=[pl.BlockSpec((1, GW), lambda i: (0, i))],
            out_specs=[pl.BlockSpec((GW, D), lambda i: (i, 0))],
            core_axis_name=("c", "s"),
            dimension_semantics=(pltpu.PARALLEL,),
            trace_scopes=False,
        )(j_hbm, g_hbm)

    return _gather


@functools.cache
def _segsum_kernel():
    @functools.partial(
        pl.kernel,
        out_type=jax.ShapeDtypeStruct((NC, N, D), jnp.float32),
        mesh=_vector_mesh(),
        scratch_types=[
            pltpu.VMEM_SHARED((N, D), jnp.float32),
            pltpu.VMEM((ZR, D), jnp.float32),
            pltpu.VMEM((GW, D), jnp.float32),
            pltpu.VMEM((GW, D), jnp.float32),
            pltpu.VMEM((1, GW), jnp.int32),
            pltpu.VMEM((1, GW), jnp.int32),
            pltpu.SemaphoreType.DMA,
            pltpu.SemaphoreType.DMA,
            pltpu.SemaphoreType.DMA,
            pltpu.SemaphoreType.DMA,
        ],
    )
    def _segsum(e_hbm, di_hbm, out_hbm, agg_sh, zbuf, eb0, eb1, ib0, ib1,
                sem0, sem1, ssem0, ssem1):
        c = lax.axis_index("c")
        s = lax.axis_index("s")
        wid = s * NC + c
        ebufs = (eb0, eb1)
        ibufs = (ib0, ib1)
        sems = (sem0, sem1)
        ssems = (ssem0, ssem1)

        @pl.loop(0, ZR)
        def _(r):
            @pl.loop(0, D, step=16)
            def _(col):
                zbuf.at[pl.ds(r, 1), pl.ds(col, 16)][...] = jnp.zeros(
                    (1, 16), jnp.float32)

        @pl.loop(s, NZCH, step=NS)
        def _(k):
            pltpu.sync_copy(zbuf, agg_sh.at[pl.ds(k * ZR, ZR)])

        plsc.subcore_barrier()

        def _start(b, k):
            pltpu.async_copy(e_hbm.at[pl.ds(k * GW, GW)], ebufs[b], sems[b])
            pltpu.async_copy(di_hbm.at[k], ibufs[b], sems[b])

        def _wait(b, k):
            pltpu.make_async_copy(
                e_hbm.at[pl.ds(k * GW, GW)], ebufs[b], sems[b]).wait()
            pltpu.make_async_copy(di_hbm.at[k], ibufs[b], sems[b]).wait()

        def _scat_start(b):
            pltpu.async_copy(ebufs[b], agg_sh.at[ibufs[b].at[0]], ssems[b],
                             add=True)

        def _scat_wait(b):
            pltpu.make_async_copy(
                ebufs[b], agg_sh.at[ibufs[b].at[0]], ssems[b]).wait()

        _start(0, wid)

        @pl.loop(0, ((NCHH + NW - 1) // NW + 1) // 2)
        def _(j):
            for b in range(2):
                j2 = 2 * j + b
                k = wid + NW * j2
                b1 = 1 - b

                @pl.when(k < NCHH)
                def _():
                    _wait(b, k)
                    _scat_start(b)

                    @pl.when(j2 >= 1)
                    def _():
                        _scat_wait(b1)

                    @pl.when(k + NW < NCHH)
                    def _():
                        _start(b1, k + NW)

        nmine = (NCHH - 1 - wid) // NW + 1
        lastb = (nmine - 1) % 2

        @pl.when(lastb == 0)
        def _():
            _scat_wait(0)

        @pl.when(lastb == 1)
        def _():
            _scat_wait(1)

        plsc.subcore_barrier()

        @pl.loop(s, NZCH, step=NS)
        def _(k):
            r0 = k * ZR
            pltpu.sync_copy(agg_sh.at[pl.ds(r0, ZR)], zbuf)
            pltpu.sync_copy(zbuf, out_hbm.at[c, pl.ds(r0, ZR)])

    return _segsum


def kernel(edge_feats, node_feats, edge_index,
           edge_W1, edge_b1, edge_W2, edge_b2, edge_ln_g, edge_ln_b,
           node_W1, node_b1, node_W2, node_b2, node_ln_g, node_ln_b):
    src = edge_index[0]
    dst = edge_index[1]
    jidxs = []
    dst3s = []
    for k in range(SPL):
        sk = src[k * EH:(k + 1) * EH]
        dk = dst[k * EH:(k + 1) * EH]
        jidxs.append(jnp.concatenate([sk, dk + N]).reshape(1, 2 * EH))
        dst3s.append(dk.reshape(NCHH, 1, GW))
    n = node_feats
    es = [None] * SPL
    for i in range(L):
        w1 = edge_W1[i]
        psd = _project(n, w1[D:].reshape(2, D, D))
        tab = psd.reshape(2 * N, D)
        ew = (w1[:D], edge_b1[i], edge_W2[i], edge_b2[i],
              edge_ln_g[i], edge_ln_b[i])
        gs = [None] * SPL
        aggps = [None] * SPL
        gs[0] = _gather_kernel()(tab, jidxs[0])
        for k in range(SPL):
            if k + 1 < SPL:
                gs[k + 1] = _gather_kernel()(tab, jidxs[k + 1])
            if i == 0:
                es[k] = _edge_mlp(edge_feats, k * EHBLK, gs[k], *ew)
            else:
                es[k] = _edge_mlp(es[k], 0, gs[k], *ew)
            aggps[k] = _segsum_kernel()(es[k], dst3s[k])
        n = _node_mlp(n, aggps, node_W1[i, :D], node_W1[i, D:],
                      node_b1[i], node_W2[i], node_b2[i],
                      node_ln_g[i], node_ln_b[i])
    return jnp.concatenate(es, axis=0), n


# speedup vs baseline: 1.0786x; 1.0786x over previous
import functools

import jax
import jax.numpy as jnp
from jax import lax
from jax.experimental import pallas as pl
from jax.experimental.pallas import tpu as pltpu
from jax.experimental.pallas import tpu_sc as plsc

L = 4
N = 10000
E = 320000
D = 128

NC = 2
NS = 16
NW = NC * NS

GW = 128
EBLK = 2000
NBLK = 1000
ZR = 80
NZCH = N // ZR
SPL = 2
EH = E // SPL
NCHH = EH // GW
EHBLK = EH // EBLK

@functools.cache
def _vector_mesh():
    return plsc.VectorSubcoreMesh(core_axis_name="c", subcore_axis_name="s")


def _bmm(a, w):
    return jnp.dot(a, w, preferred_element_type=jnp.float32)


def _project_body(n_ref, w_ref, out_ref):
    out_ref[0] = _bmm(n_ref[...], w_ref[0])


def _project(n, wsd):
    return pl.pallas_call(
        _project_body,
        grid=(2, N // NBLK),
        in_specs=[
            pl.BlockSpec((NBLK, D), lambda t, i: (i, 0)),
            pl.BlockSpec((1, D, D), lambda t, i: (t, 0, 0)),
        ],
        out_specs=pl.BlockSpec((1, NBLK, D), lambda t, i: (t, i, 0)),
        out_shape=jax.ShapeDtypeStruct((2, N, D), jnp.float32),
    )(n, wsd)


def _edge_body(e_ref, g1_ref, g2_ref, w1_ref, b1_ref, w2_ref, b2_ref,
               lng_ref, lnb_ref, o_ref):
    e = e_ref[...]
    pre = (_bmm(e, w1_ref[...])
           + g1_ref[...] + g2_ref[...] + b1_ref[...])
    h1 = pre * jax.nn.sigmoid(pre)
    h = _bmm(h1, w2_ref[...]) + b2_ref[...]
    m = jnp.mean(h, axis=-1, keepdims=True)
    v = jnp.mean((h - m) * (h - m), axis=-1, keepdims=True)
    hn = (h - m) * lax.rsqrt(v + 1e-5) * lng_ref[...] + lnb_ref[...]
    o_ref[...] = e + hn


def _edge_mlp(e, e_off, g, w1, b1, w2, b2, lng, lnb):
    row = lambda i: (i, 0)
    full = lambda i: (0, 0)
    return pl.pallas_call(
        _edge_body,
        grid=(EHBLK,),
        in_specs=[
            pl.BlockSpec((EBLK, D), lambda i: (e_off + i, 0)),
            pl.BlockSpec((EBLK, D), row),
            pl.BlockSpec((EBLK, D), lambda i: (EHBLK + i, 0)),
            pl.BlockSpec((D, D), full),
            pl.BlockSpec((1, D), full),
            pl.BlockSpec((D, D), full),
            pl.BlockSpec((1, D), full),
            pl.BlockSpec((1, D), full),
            pl.BlockSpec((1, D), full),
        ],
        out_specs=pl.BlockSpec((EBLK, D), row),
        out_shape=jax.ShapeDtypeStruct((EH, D), jnp.float32),
    )(e, g, g, w1, b1.reshape(1, D), w2, b2.reshape(1, D),
      lng.reshape(1, D), lnb.reshape(1, D))


def _node_body(n_ref, *rest):
    (a_refs, (w1a_ref, w1b_ref, b1_ref, w2_ref, b2_ref, lng_ref, lnb_ref,
              o_ref)) = rest[:2 * SPL], rest[2 * SPL:]
    n = n_ref[...]
    agg = a_refs[0][0]
    for r in a_refs[1:]:
        agg = agg + r[0]
    pre = (_bmm(n, w1a_ref[...]) + _bmm(agg, w1b_ref[...]) + b1_ref[...])
    h1 = pre * jax.nn.sigmoid(pre)
    h = _bmm(h1, w2_ref[...]) + b2_ref[...]
    m = jnp.mean(h, axis=-1, keepdims=True)
    v = jnp.mean((h - m) * (h - m), axis=-1, keepdims=True)
    hn = (h - m) * lax.rsqrt(v + 1e-5) * lng_ref[...] + lnb_ref[...]
    o_ref[...] = n + hn


def _node_mlp(n, aggps, w1a, w1b, b1, w2, b2, lng, lnb):
    row = lambda i: (i, 0)
    full = lambda i: (0, 0)
    p0 = lambda i: (0, i, 0)
    p1 = lambda i: (1, i, 0)
    agg_specs = []
    agg_args = []
    for a in aggps:
        agg_specs += [pl.BlockSpec((1, NBLK, D), p0),
                      pl.BlockSpec((1, NBLK, D), p1)]
        agg_args += [a, a]
    return pl.pallas_call(
        _node_body,
        grid=(N // NBLK,),
        in_specs=[pl.BlockSpec((NBLK, D), row)] + agg_specs + [
            pl.BlockSpec((D, D), full),
            pl.BlockSpec((D, D), full),
            pl.BlockSpec((1, D), full),
            pl.BlockSpec((D, D), full),
            pl.BlockSpec((1, D), full),
            pl.BlockSpec((1, D), full),
            pl.BlockSpec((1, D), full),
        ],
        out_specs=pl.BlockSpec((NBLK, D), row),
        out_shape=jax.ShapeDtypeStruct((N, D), jnp.float32),
    )(n, *agg_args, w1a, w1b, b1.reshape(1, D), w2,
      b2.reshape(1, D), lng.reshape(1, D), lnb.reshape(1, D))


NCH2 = 2 * EH // GW


@functools.cache
def _gather_kernel():
    @functools.partial(
        pl.kernel,
        out_type=jax.ShapeDtypeStruct((2 * EH, D), jnp.float32),
        mesh=_vector_mesh(),
        scratch_types=[
            pltpu.VMEM((GW, D), jnp.float32),
            pltpu.VMEM((GW, D), jnp.float32),
            pltpu.VMEM((GW, D), jnp.float32),
            pltpu.VMEM((1, GW), jnp.int32),
            pltpu.VMEM((1, GW), jnp.int32),
            pltpu.VMEM((1, GW), jnp.int32),
            pltpu.SemaphoreType.DMA,
            pltpu.SemaphoreType.DMA,
            pltpu.SemaphoreType.DMA,
            pltpu.SemaphoreType.DMA,
            pltpu.SemaphoreType.DMA,
            pltpu.SemaphoreType.DMA,
            pltpu.SemaphoreType.DMA,
            pltpu.SemaphoreType.DMA,
            pltpu.SemaphoreType.DMA,
        ],
    )
    def _gather(tab_hbm, j_hbm, g_hbm,
                gb0, gb1, gb2, ib0, ib1, ib2,
                is0, is1, is2, gs0, gs1, gs2, ws0, ws1, ws2):
        c = lax.axis_index("c")
        s = lax.axis_index("s")
        wid = s * NC + c
        gbufs = (gb0, gb1, gb2)
        ibufs = (ib0, ib1, ib2)
        isems = (is0, is1, is2)
        gsems = (gs0, gs1, gs2)
        wsems = (ws0, ws1, ws2)

        def k_of(j):
            return wid + NW * j

        def idx_start(b, j):
            pltpu.async_copy(j_hbm.at[k_of(j)], ibufs[b], isems[b])

        def idx_wait(b, j):
            pltpu.make_async_copy(j_hbm.at[k_of(j)], ibufs[b],
                                  isems[b]).wait()

        def g_start(b):
            pltpu.async_copy(tab_hbm.at[ibufs[b].at[0]], gbufs[b], gsems[b])

        def g_wait(b):
            pltpu.make_async_copy(tab_hbm.at[ibufs[b].at[0]], gbufs[b],
                                  gsems[b]).wait()

        def w_start(b, j):
            pltpu.async_copy(gbufs[b], g_hbm.at[pl.ds(k_of(j) * GW, GW)],
                             wsems[b])

        def w_wait(b, j):
            pltpu.make_async_copy(gbufs[b],
                                  g_hbm.at[pl.ds(k_of(j) * GW, GW)],
                                  wsems[b]).wait()

        for b in range(3):
            @pl.when(k_of(b) < NCH2)
            def _():
                idx_start(b, b)

        NSLOT = (NCH2 + NW - 1) // NW

        @pl.loop(0, (NSLOT + 1 + 2) // 3)
        def _(jj):
            for b in range(3):
                j = 3 * jj + b
                k = wid + NW * j
                bp = (b + 2) % 3

                @pl.when(k < NCH2)
                def _():
                    @pl.when(j >= 3)
                    def _():
                        w_wait(b, j - 3)

                    idx_wait(b, j)
                    g_start(b)

                @pl.when((j >= 1) & (k - NW < NCH2))
                def _():
                    g_wait(bp)
                    w_start(bp, j - 1)

                    @pl.when(k_of(j + 2) < NCH2)
                    def _():
                        idx_start(bp, j + 2)

        nmine = (NCH2 - 1 - wid) // NW + 1
        for b in range(3):
            @pl.when(nmine > b)
            def _():
                jb = nmine - 1 - lax.rem(nmine - 1 - b, 3)
                w_wait(b, jb)

    return _gather


@functools.cache
def _segsum_kernel():
    @functools.partial(
        pl.kernel,
        out_type=jax.ShapeDtypeStruct((NC, N, D), jnp.float32),
        mesh=_vector_mesh(),
        scratch_types=[
            pltpu.VMEM_SHARED((N, D), jnp.float32),
            pltpu.VMEM((ZR, D), jnp.float32),
            pltpu.VMEM((GW, D), jnp.float32),
            pltpu.VMEM((GW, D), jnp.float32),
            pltpu.VMEM((1, GW), jnp.int32),
            pltpu.VMEM((1, GW), jnp.int32),
            pltpu.SemaphoreType.DMA,
            pltpu.SemaphoreType.DMA,
            pltpu.SemaphoreType.DMA,
            pltpu.SemaphoreType.DMA,
        ],
    )
    def _segsum(e_hbm, di_hbm, out_hbm, agg_sh, zbuf, eb0, eb1, ib0, ib1,
                sem0, sem1, ssem0, ssem1):
        c = lax.axis_index("c")
        s = lax.axis_index("s")
        wid = s * NC + c
        ebufs = (eb0, eb1)
        ibufs = (ib0, ib1)
        sems = (sem0, sem1)
        ssems = (ssem0, ssem1)

        @pl.loop(0, ZR)
        def _(r):
            @pl.loop(0, D, step=16)
            def _(col):
                zbuf.at[pl.ds(r, 1), pl.ds(col, 16)][...] = jnp.zeros(
                    (1, 16), jnp.float32)

        @pl.loop(s, NZCH, step=NS)
        def _(k):
            pltpu.sync_copy(zbuf, agg_sh.at[pl.ds(k * ZR, ZR)])

        plsc.subcore_barrier()

        def _start(b, k):
            pltpu.async_copy(e_hbm.at[pl.ds(k * GW, GW)], ebufs[b], sems[b])
            pltpu.async_copy(di_hbm.at[k], ibufs[b], sems[b])

        def _wait(b, k):
            pltpu.make_async_copy(
                e_hbm.at[pl.ds(k * GW, GW)], ebufs[b], sems[b]).wait()
            pltpu.make_async_copy(di_hbm.at[k], ibufs[b], sems[b]).wait()

        def _scat_start(b):
            pltpu.async_copy(ebufs[b], agg_sh.at[ibufs[b].at[0]], ssems[b],
                             add=True)

        def _scat_wait(b):
            pltpu.make_async_copy(
                ebufs[b], agg_sh.at[ibufs[b].at[0]], ssems[b]).wait()

        _start(0, wid)

        @pl.loop(0, ((NCHH + NW - 1) // NW + 1) // 2)
        def _(j):
            for b in range(2):
                j2 = 2 * j + b
                k = wid + NW * j2
                b1 = 1 - b

                @pl.when(k < NCHH)
                def _():
                    _wait(b, k)
                    _scat_start(b)

                    @pl.when(j2 >= 1)
                    def _():
                        _scat_wait(b1)

                    @pl.when(k + NW < NCHH)
                    def _():
                        _start(b1, k + NW)

        nmine = (NCHH - 1 - wid) // NW + 1
        lastb = (nmine - 1) % 2

        @pl.when(lastb == 0)
        def _():
            _scat_wait(0)

        @pl.when(lastb == 1)
        def _():
            _scat_wait(1)

        plsc.subcore_barrier()

        @pl.loop(s, NZCH, step=NS)
        def _(k):
            r0 = k * ZR
            pltpu.sync_copy(agg_sh.at[pl.ds(r0, ZR)], zbuf)
            pltpu.sync_copy(zbuf, out_hbm.at[c, pl.ds(r0, ZR)])

    return _segsum


def kernel(edge_feats, node_feats, edge_index,
           edge_W1, edge_b1, edge_W2, edge_b2, edge_ln_g, edge_ln_b,
           node_W1, node_b1, node_W2, node_b2, node_ln_g, node_ln_b):
    src = edge_index[0]
    dst = edge_index[1]
    jidxs = []
    dst3s = []
    for k in range(SPL):
        sk = src[k * EH:(k + 1) * EH]
        dk = dst[k * EH:(k + 1) * EH]
        jidxs.append(jnp.concatenate([sk, dk + N]).reshape(NCH2, 1, GW))
        dst3s.append(dk.reshape(NCHH, 1, GW))
    n = node_feats
    es = [None] * SPL
    for i in range(L):
        w1 = edge_W1[i]
        psd = _project(n, w1[D:].reshape(2, D, D))
        tab = psd.reshape(2 * N, D)
        ew = (w1[:D], edge_b1[i], edge_W2[i], edge_b2[i],
              edge_ln_g[i], edge_ln_b[i])
        gs = [None] * SPL
        aggps = [None] * SPL
        gs[0] = _gather_kernel()(tab, jidxs[0])
        for k in range(SPL):
            if k + 1 < SPL:
                gs[k + 1] = _gather_kernel()(tab, jidxs[k + 1])
            if i == 0:
                es[k] = _edge_mlp(edge_feats, k * EHBLK, gs[k], *ew)
            else:
                es[k] = _edge_mlp(es[k], 0, gs[k], *ew)
            aggps[k] = _segsum_kernel()(es[k], dst3s[k])
        n = _node_mlp(n, aggps, node_W1[i, :D], node_W1[i, D:],
                      node_b1[i], node_W2[i], node_b2[i],
                      node_ln_g[i], node_ln_b[i])
    return jnp.concatenate(es, axis=0), n
